# Initial kernel scaffold; baseline (speedup 1.0000x reference)
import jax
import jax.numpy as jnp
from jax.experimental import pallas as pl


def kernel(x, single, emb):
    raise NotImplementedError("write your pallas kernel here")


# speedup vs baseline: 1.1046x; 1.1046x over previous
import functools

import jax
import jax.numpy as jnp
from jax import lax
from jax.experimental import pallas as pl
from jax.experimental.pallas import tpu as pltpu
from jax.experimental.pallas import tpu_sc as plsc

VOCAB = 1000000
EMB = 32
B_TOTAL = 16384 * 50
NUM_WORKERS = 32
B_PER_W = B_TOTAL // NUM_WORKERS
CHUNK = 128
K_INFLIGHT = 8
ROWS_PER_BUF = CHUNK * K_INFLIGHT
N_STEPS = B_PER_W // ROWS_PER_BUF
N_CHUNKS = B_PER_W // CHUNK


def _gather_kernel(x_hbm, emb_hbm, out_hbm, idx_v, rows_v, gsem):
    wid = lax.axis_index("s") * 2 + lax.axis_index("c")
    base = wid * B_PER_W
    pltpu.sync_copy(x_hbm.at[wid], idx_v)

    def step(g, _):
        copies = []
        for j in range(K_INFLIGHT):
            row = g * K_INFLIGHT + j
            copies.append(
                pltpu.async_copy(
                    emb_hbm.at[idx_v.at[row]],
                    rows_v.at[pl.ds(j * CHUNK, CHUNK)],
                    gsem,
                )
            )
        for c in copies:
            c.wait()
        pltpu.sync_copy(
            rows_v,
            out_hbm.at[pl.ds(base + g * ROWS_PER_BUF, ROWS_PER_BUF)],
        )
        return 0

    lax.fori_loop(0, N_STEPS, step, 0)


@jax.jit
def _embed_lookup(x2, emb):
    mesh = plsc.VectorSubcoreMesh(core_axis_name="c", subcore_axis_name="s")
    run = functools.partial(
        pl.kernel,
        mesh=mesh,
        out_type=jax.ShapeDtypeStruct((B_TOTAL, EMB), jnp.float32),
        scratch_types=[
            pltpu.VMEM((N_CHUNKS, CHUNK), jnp.int32),
            pltpu.VMEM((ROWS_PER_BUF, EMB), jnp.float32),
            pltpu.SemaphoreType.DMA,
        ],
        compiler_params=pltpu.CompilerParams(use_tc_tiling_on_sc=False),
    )(_gather_kernel)
    return run(x2, emb)


def kernel(x, single, emb):
    idx = (x * jnp.asarray(single, dtype=x.dtype)).astype(jnp.int32)
    x2 = idx.reshape(NUM_WORKERS, N_CHUNKS, CHUNK)
    out = _embed_lookup(x2, emb)
    return out.reshape(x.shape[0], x.shape[1], EMB)


# speedup vs baseline: 1.1112x; 1.0060x over previous
import functools

import jax
import jax.numpy as jnp
from jax import lax
from jax.experimental import pallas as pl
from jax.experimental.pallas import tpu as pltpu
from jax.experimental.pallas import tpu_sc as plsc

VOCAB = 1000000
EMB = 32
B_TOTAL = 16384 * 50
NUM_WORKERS = 32
B_PER_W = B_TOTAL // NUM_WORKERS
CHUNK = 128
K_INFLIGHT = 10
ROWS_PER_BUF = CHUNK * K_INFLIGHT
NBUF = 2
N_STEPS = B_PER_W // ROWS_PER_BUF
N_CHUNKS = B_PER_W // CHUNK


def _gather_kernel(x_hbm, emb_hbm, out_hbm, idx_v, rows0, rows1, gsem0, gsem1):
    wid = lax.axis_index("s") * 2 + lax.axis_index("c")
    base = wid * B_PER_W
    pltpu.sync_copy(x_hbm.at[wid], idx_v)

    rows = (rows0, rows1)
    gsems = (gsem0, gsem1)

    def issue(g, b):
        for j in range(K_INFLIGHT):
            pltpu.async_copy(
                emb_hbm.at[idx_v.at[g * K_INFLIGHT + j]],
                rows[b].at[pl.ds(j * CHUNK, CHUNK)],
                gsems[b],
            )

    def drain(b):
        pltpu.make_async_copy(
            emb_hbm.at[pl.ds(0, ROWS_PER_BUF)],
            rows[b],
            gsems[b],
        ).wait()

    issue(0, 0)

    def pair(t, _):
        for b in range(NBUF):
            g = t * NBUF + b
            drain(b)

            @pl.when(g + 1 < N_STEPS)
            def _():
                issue(g + 1, 1 - b)

            pltpu.sync_copy(
                rows[b],
                out_hbm.at[pl.ds(base + g * ROWS_PER_BUF, ROWS_PER_BUF)],
            )
        return 0

    lax.fori_loop(0, N_STEPS // NBUF, pair, 0)


@jax.jit
def _embed_lookup(x2, emb):
    mesh = plsc.VectorSubcoreMesh(core_axis_name="c", subcore_axis_name="s")
    run = functools.partial(
        pl.kernel,
        mesh=mesh,
        out_type=jax.ShapeDtypeStruct((B_TOTAL, EMB), jnp.float32),
        scratch_types=[
            pltpu.VMEM((N_CHUNKS, CHUNK), jnp.int32),
            pltpu.VMEM((ROWS_PER_BUF, EMB), jnp.float32),
            pltpu.VMEM((ROWS_PER_BUF, EMB), jnp.float32),
            pltpu.SemaphoreType.DMA,
            pltpu.SemaphoreType.DMA,
        ],
        compiler_params=pltpu.CompilerParams(use_tc_tiling_on_sc=False),
    )(_gather_kernel)
    return run(x2, emb)


def kernel(x, single, emb):
    idx = (x * jnp.asarray(single, dtype=x.dtype)).astype(jnp.int32)
    x2 = idx.reshape(NUM_WORKERS, N_CHUNKS, CHUNK)
    out = _embed_lookup(x2, emb)
    return out.reshape(x.shape[0], x.shape[1], EMB)


# speedup vs baseline: 1.3930x; 1.2535x over previous
import functools

import jax
import jax.numpy as jnp
from jax import lax
from jax.experimental import pallas as pl
from jax.experimental.pallas import tpu as pltpu
from jax.experimental.pallas import tpu_sc as plsc

VOCAB = 1000000
EMB = 32
SEQ = 50
BATCH = 16384
NUM_WORKERS = 32
BCH = 512
NB_C = BATCH // BCH
N_BLOCKS = SEQ * NB_C
BLK_PER_TILE = N_BLOCKS // NUM_WORKERS
CHUNK = 128
K_PER_BLK = BCH // CHUNK


def _gather_kernel(xT_hbm, emb_hbm, out_hbm,
                   idx_v, rows0, rows1, tout0, tout1,
                   gsem0, gsem1, wsem0, wsem1):
    wid = lax.axis_index("s") * 2 + lax.axis_index("c")
    b0 = wid * BLK_PER_TILE
    iota = lax.iota(jnp.int32, 16)
    rows = (rows0, rows1)
    touts = (tout0, tout1)
    gsems = (gsem0, gsem1)
    wsems = (wsem0, wsem1)

    def load_idx(bi, buf):
        s = bi // NB_C
        c = bi % NB_C
        for q in range(K_PER_BLK):
            pltpu.sync_copy(
                xT_hbm.at[s, pl.ds(c * BCH + q * CHUNK, CHUNK)],
                idx_v.at[K_PER_BLK * buf + q],
            )

    def fire(buf):
        for q in range(K_PER_BLK):
            pltpu.async_copy(
                emb_hbm.at[idx_v.at[K_PER_BLK * buf + q]],
                rows[buf].at[pl.ds(q * CHUNK, CHUNK)],
                gsems[buf],
            )

    def drain_g(buf):
        pltpu.make_async_copy(
            emb_hbm.at[pl.ds(0, BCH)], rows[buf], gsems[buf]
        ).wait()

    def transpose(buf):
        def fbody(f, _):
            fv = jnp.full((16,), f, jnp.int32)
            for j in range(BCH // 16):
                v = plsc.load_gather(rows[buf], [iota + 16 * j, fv])
                touts[buf][f, pl.ds(16 * j, 16)] = v
            return 0

        lax.fori_loop(0, EMB, fbody, 0)

    def write(bi, buf):
        s = bi // NB_C
        c = bi % NB_C
        pltpu.async_copy(
            touts[buf], out_hbm.at[s, :, pl.ds(c * BCH, BCH)], wsems[buf]
        )

    def drain_w(buf):
        pltpu.make_async_copy(
            touts[buf], out_hbm.at[0, :, pl.ds(0, BCH)], wsems[buf]
        ).wait()

    load_idx(b0, 0)
    fire(0)

    def pair(t, _):
        for buf in range(2):
            k = 2 * t + buf
            bi = b0 + k
            drain_g(buf)

            @pl.when(k + 1 < BLK_PER_TILE)
            def _():
                load_idx(bi + 1, 1 - buf)
                fire(1 - buf)

            @pl.when(k >= 2)
            def _():
                drain_w(buf)

            transpose(buf)
            write(bi, buf)
        return 0

    lax.fori_loop(0, BLK_PER_TILE // 2, pair, 0)
    drain_w(0)
    drain_w(1)


@jax.jit
def _embed_lookup(xT, emb):
    mesh = plsc.VectorSubcoreMesh(core_axis_name="c", subcore_axis_name="s")
    run = functools.partial(
        pl.kernel,
        mesh=mesh,
        out_type=jax.ShapeDtypeStruct((SEQ, EMB, BATCH), jnp.float32),
        scratch_types=[
            pltpu.VMEM((2 * K_PER_BLK, CHUNK), jnp.int32),
            pltpu.VMEM((BCH, EMB), jnp.float32),
            pltpu.VMEM((BCH, EMB), jnp.float32),
            pltpu.VMEM((EMB, BCH), jnp.float32),
            pltpu.VMEM((EMB, BCH), jnp.float32),
            pltpu.SemaphoreType.DMA,
            pltpu.SemaphoreType.DMA,
            pltpu.SemaphoreType.DMA,
            pltpu.SemaphoreType.DMA,
        ],
        compiler_params=pltpu.CompilerParams(
            use_tc_tiling_on_sc=False, needs_layout_passes=False
        ),
    )(_gather_kernel)
    return run(xT, emb)


def kernel(x, single, emb):
    idx = (x * jnp.asarray(single, dtype=x.dtype)).astype(jnp.int32)
    out = _embed_lookup(idx.T, emb)
    return jnp.transpose(out, (2, 0, 1))


# speedup vs baseline: 1.6935x; 1.2157x over previous
import functools

import jax
import jax.numpy as jnp
from jax import lax
from jax.experimental import pallas as pl
from jax.experimental.pallas import tpu as pltpu
from jax.experimental.pallas import tpu_sc as plsc

VOCAB = 1000000
EMB = 32
SEQ = 50
BATCH = 16384
NUM_WORKERS = 32
BCH = 512
NB_C = BATCH // BCH
N_BLOCKS = SEQ * NB_C
BLK_PER_TILE = N_BLOCKS // NUM_WORKERS
CHUNK = 128
K_PER_BLK = BCH // CHUNK


def _gather_kernel(xT_hbm, emb_hbm, out_hbm,
                   idx_v, rows0, rows1, tout0, tout1,
                   gsem0, gsem1, wsem0, wsem1):
    wid = lax.axis_index("s") * 2 + lax.axis_index("c")
    b0 = wid * BLK_PER_TILE
    iota = lax.iota(jnp.int32, 16)
    rows = (rows0, rows1)
    touts = (tout0, tout1)
    gsems = (gsem0, gsem1)
    wsems = (wsem0, wsem1)

    def load_idx(bi, buf):
        s = bi // NB_C
        c = bi % NB_C
        for q in range(K_PER_BLK):
            pltpu.sync_copy(
                xT_hbm.at[s, pl.ds(c * BCH + q * CHUNK, CHUNK)],
                idx_v.at[K_PER_BLK * buf + q],
            )

    def fire(buf):
        for q in range(K_PER_BLK):
            pltpu.async_copy(
                emb_hbm.at[idx_v.at[K_PER_BLK * buf + q]],
                rows[buf].at[pl.ds(q * CHUNK, CHUNK)],
                gsems[buf],
            )

    def drain_g(buf):
        pltpu.make_async_copy(
            emb_hbm.at[pl.ds(0, BCH)], rows[buf], gsems[buf]
        ).wait()

    fcols = [jnp.full((16,), f, jnp.int32) for f in range(EMB)]

    def transpose(buf):
        @plsc.parallel_loop(0, BCH // 16, unroll=2)
        def _(j):
            row0 = iota + 16 * j
            off = 16 * j
            for f in range(EMB):
                v = plsc.load_gather(rows[buf], [row0, fcols[f]])
                touts[buf][f, pl.ds(off, 16)] = v

    def write(bi, buf):
        s = bi // NB_C
        c = bi % NB_C
        pltpu.async_copy(
            touts[buf], out_hbm.at[s, :, pl.ds(c * BCH, BCH)], wsems[buf]
        )

    def drain_w(buf):
        pltpu.make_async_copy(
            touts[buf], out_hbm.at[0, :, pl.ds(0, BCH)], wsems[buf]
        ).wait()

    load_idx(b0, 0)
    fire(0)

    def pair(t, _):
        for buf in range(2):
            k = 2 * t + buf
            bi = b0 + k
            drain_g(buf)

            @pl.when(k + 1 < BLK_PER_TILE)
            def _():
                load_idx(bi + 1, 1 - buf)
                fire(1 - buf)

            @pl.when(k >= 2)
            def _():
                drain_w(buf)

            transpose(buf)
            write(bi, buf)
        return 0

    lax.fori_loop(0, BLK_PER_TILE // 2, pair, 0)
    drain_w(0)
    drain_w(1)


@jax.jit
def _embed_lookup(xT, emb):
    mesh = plsc.VectorSubcoreMesh(core_axis_name="c", subcore_axis_name="s")
    run = functools.partial(
        pl.kernel,
        mesh=mesh,
        out_type=jax.ShapeDtypeStruct((SEQ, EMB, BATCH), jnp.float32),
        scratch_types=[
            pltpu.VMEM((2 * K_PER_BLK, CHUNK), jnp.int32),
            pltpu.VMEM((BCH, EMB), jnp.float32),
            pltpu.VMEM((BCH, EMB), jnp.float32),
            pltpu.VMEM((EMB, BCH), jnp.float32),
            pltpu.VMEM((EMB, BCH), jnp.float32),
            pltpu.SemaphoreType.DMA,
            pltpu.SemaphoreType.DMA,
            pltpu.SemaphoreType.DMA,
            pltpu.SemaphoreType.DMA,
        ],
        compiler_params=pltpu.CompilerParams(
            use_tc_tiling_on_sc=False,
            needs_layout_passes=False,
            disable_bounds_checks=True,
        ),
    )(_gather_kernel)
    return run(xT, emb)


def kernel(x, single, emb):
    idx = (x * jnp.asarray(single, dtype=x.dtype)).astype(jnp.int32)
    out = _embed_lookup(idx.T, emb)
    return jnp.transpose(out, (2, 0, 1))
